# baseline (speedup 1.0000x reference)
import functools

import jax
import jax.numpy as jnp
from jax.experimental import pallas as pl
from jax.experimental.pallas import tpu as pltpu

B, N, F, H = 2, 4096, 128, 128
BM = 512
NI = N // BM


def _decode(t):
    in_a = t < NI
    in_b = (t >= NI) & (t < 3 * NI)
    j = (t - NI) // 2
    r = (t - NI) % 2
    b = jnp.where(in_a, 0, jnp.where(in_b, r, 1))
    p = jnp.where(in_a, 0, jnp.where(in_b, 1 - r, 1))
    i = jnp.where(in_a, t, jnp.where(in_b, j, t - 3 * NI))
    return b, p, i


def _adj_index(t):
    in_a = t < NI
    in_b = (t >= NI) & (t < 3 * NI)
    j = (t - NI) // 2
    r = (t - NI) % 2
    b_sel = jnp.where(
        in_a, 0,
        jnp.where(in_b, jnp.where(r == 1, 1, jnp.where(j == 0, 0, 1)), 1),
    )
    i_sel = jnp.where(
        in_a, t,
        jnp.where(
            in_b,
            jnp.where(r == 1, j, jnp.where(j == 0, NI - 1, j - 1)),
            NI - 1,
        ),
    )
    return (b_sel, i_sel, 0)


def _h2_index(t):
    in_a = t < NI
    in_b = (t >= NI) & (t < 3 * NI)
    j = (t - NI) // 2
    b_sel = jnp.where(t < 3 * NI, 0, 1)
    i_sel = jnp.where(in_a, 0, jnp.where(in_b, j, t - 3 * NI))
    return (b_sel, i_sel, 0)


def _fused_kernel(adj_ref, x_ref, w1_ref, b1_ref, w2_ref, b2_ref,
                  h2_ref, gh_ref, s1_scr, s2_scr, cache_scr):
    t = pl.program_id(0)
    b, p, i = _decode(t)

    @pl.when(p == 0)
    def _():
        @pl.when(i == 0)
        def _():
            s1 = jnp.dot(
                x_ref[0], w1_ref[...], preferred_element_type=jnp.float32
            )
            s1_scr[...] = s1.astype(jnp.bfloat16)

        a = adj_ref[0].astype(jnp.bfloat16)
        cache_scr[pl.ds(i * BM, BM), :] = a
        tacc = jnp.dot(a, s1_scr[...], preferred_element_type=jnp.float32)
        h1 = jnp.maximum(tacc + b1_ref[...], 0.0)
        s2_scr[b, pl.ds(i * BM, BM), :] = jnp.dot(
            h1, w2_ref[...], preferred_element_type=jnp.float32
        ).astype(jnp.bfloat16)
        part = jnp.sum(h1, axis=0, keepdims=True)
        row = 2 * b

        @pl.when(i == 0)
        def _():
            gh_ref[pl.ds(row, 1), :] = part

        @pl.when(i != 0)
        def _():
            gh_ref[pl.ds(row, 1), :] += part

    @pl.when(p == 1)
    def _():
        a = cache_scr[pl.ds(i * BM, BM), :]
        tacc = jnp.dot(a, s2_scr[b], preferred_element_type=jnp.float32)
        h2 = jnp.maximum(tacc + b2_ref[...], 0.0)
        h2_ref[...] = h2[None]
        part = jnp.sum(h2, axis=0, keepdims=True)
        row = 2 * b + 1

        @pl.when(i == 0)
        def _():
            gh_ref[pl.ds(row, 1), :] = part

        @pl.when(i != 0)
        def _():
            gh_ref[pl.ds(row, 1), :] += part


@functools.partial(jax.jit, static_argnames=("interpret",))
def _encoder(x, adj, W1, b1, W2, b2, interpret=False):
    b1r = b1.reshape(1, H)
    b2r = b2.reshape(1, H)

    h2, gh = pl.pallas_call(
        _fused_kernel,
        grid=(4 * NI,),
        in_specs=[
            pl.BlockSpec((1, BM, N), _adj_index),
            pl.BlockSpec((1, N, F), lambda t: (jnp.where(t < NI, 0, 1), 0, 0)),
            pl.BlockSpec((F, H), lambda t: (0, 0)),
            pl.BlockSpec((1, H), lambda t: (0, 0)),
            pl.BlockSpec((H, H), lambda t: (0, 0)),
            pl.BlockSpec((1, H), lambda t: (0, 0)),
        ],
        out_specs=[
            pl.BlockSpec((1, BM, H), _h2_index),
            pl.BlockSpec((4, H), lambda t: (0, 0)),
        ],
        out_shape=[
            jax.ShapeDtypeStruct((B, N, H), jnp.float32),
            jax.ShapeDtypeStruct((4, H), jnp.float32),
        ],
        scratch_shapes=[
            pltpu.VMEM((N, H), jnp.bfloat16),
            pltpu.VMEM((B, N, H), jnp.bfloat16),
            pltpu.VMEM((N, N), jnp.bfloat16),
        ],
        compiler_params=pltpu.CompilerParams(
            dimension_semantics=("arbitrary",),
            vmem_limit_bytes=100 * 1024 * 1024,
        ),
        interpret=interpret,
    )(adj, x, W1, b1r, W2, b2r)

    return h2, gh.reshape(B, 2 * H)


def kernel(x, adj, W1, b1, W2, b2):
    return _encoder(x, adj, W1, b1, W2, b2)


# speedup vs baseline: 1.1031x; 1.1031x over previous
import functools

import jax
import jax.numpy as jnp
from jax.experimental import pallas as pl
from jax.experimental.pallas import tpu as pltpu

B, N, F, H = 2, 4096, 128, 128
BM = 512
NUM_I = N // BM


def _fused_kernel(adj_ref, x_ref, w1_ref, b1_ref, w2_ref, b2_ref,
                  h2_ref, gh_ref, s1_scr, s2_scr, cache_scr):
    p = pl.program_id(1)
    i = pl.program_id(2)

    @pl.when((p == 0) & (i == 0))
    def _():
        s1 = jnp.dot(x_ref[0], w1_ref[...], preferred_element_type=jnp.float32)
        s1_scr[...] = s1.astype(jnp.bfloat16)

    @pl.when(p == 0)
    def _():
        a = adj_ref[0].astype(jnp.bfloat16)
        cache_scr[pl.ds(i * BM, BM), :] = a
        t = jnp.dot(a, s1_scr[...], preferred_element_type=jnp.float32)
        h1 = jnp.maximum(t + b1_ref[...], 0.0)
        s2_scr[pl.ds(i * BM, BM), :] = jnp.dot(
            h1, w2_ref[...], preferred_element_type=jnp.float32
        ).astype(jnp.bfloat16)
        gh_part = jnp.sum(h1, axis=0, keepdims=True)[None, None]

        @pl.when(i == 0)
        def _():
            gh_ref[...] = gh_part

        @pl.when(i != 0)
        def _():
            gh_ref[...] += gh_part

    @pl.when(p == 1)
    def _():
        a = cache_scr[pl.ds(i * BM, BM), :]
        t = jnp.dot(a, s2_scr[...], preferred_element_type=jnp.float32)
        h2 = jnp.maximum(t + b2_ref[...], 0.0)
        h2_ref[...] = h2[None]
        gh_part = jnp.sum(h2, axis=0, keepdims=True)[None, None]

        @pl.when(i == 0)
        def _():
            gh_ref[...] = gh_part

        @pl.when(i != 0)
        def _():
            gh_ref[...] += gh_part


@functools.partial(jax.jit, static_argnames=("interpret",))
def _encoder(x, adj, W1, b1, W2, b2, interpret=False):
    b1r = b1.reshape(1, H)
    b2r = b2.reshape(1, H)

    h2, gh = pl.pallas_call(
        _fused_kernel,
        grid=(B, 2, NUM_I),
        in_specs=[
            pl.BlockSpec(
                (1, BM, N),
                lambda b, p, i: (b, jnp.where(p == 0, i, NUM_I - 1), 0),
            ),
            pl.BlockSpec((1, N, F), lambda b, p, i: (b, 0, 0)),
            pl.BlockSpec((F, H), lambda b, p, i: (0, 0)),
            pl.BlockSpec((1, H), lambda b, p, i: (0, 0)),
            pl.BlockSpec((H, H), lambda b, p, i: (0, 0)),
            pl.BlockSpec((1, H), lambda b, p, i: (0, 0)),
        ],
        out_specs=[
            pl.BlockSpec(
                (1, BM, H),
                lambda b, p, i: (b, jnp.where(p == 0, 0, i), 0),
            ),
            pl.BlockSpec((1, 1, 1, H), lambda b, p, i: (b, p, 0, 0)),
        ],
        out_shape=[
            jax.ShapeDtypeStruct((B, N, H), jnp.float32),
            jax.ShapeDtypeStruct((B, 2, 1, H), jnp.float32),
        ],
        scratch_shapes=[
            pltpu.VMEM((N, H), jnp.bfloat16),
            pltpu.VMEM((N, H), jnp.bfloat16),
            pltpu.VMEM((N, N), jnp.bfloat16),
        ],
        compiler_params=pltpu.CompilerParams(
            dimension_semantics=("parallel", "arbitrary", "arbitrary"),
            vmem_limit_bytes=100 * 1024 * 1024,
        ),
        interpret=interpret,
    )(adj, x, W1, b1r, W2, b2r)

    return h2, gh.reshape(B, 2 * H)


def kernel(x, adj, W1, b1, W2, b2):
    return _encoder(x, adj, W1, b1, W2, b2)


# speedup vs baseline: 1.1530x; 1.0452x over previous
import functools

import jax
import jax.numpy as jnp
from jax.experimental import pallas as pl
from jax.experimental.pallas import tpu as pltpu

B, N, F, H = 2, 4096, 128, 128
BM = 512
NI = N // BM
RING = 3
NF = B * NI


def _start_fetch(adj_ref, ring_scr, sem, f):
    fb = f // NI
    fi = f % NI
    pltpu.make_async_copy(
        adj_ref.at[fb, pl.ds(fi * BM, BM), :],
        ring_scr.at[f % RING],
        sem.at[f % RING],
    ).start()


def _wait_fetch(adj_ref, ring_scr, sem, f):
    fb = f // NI
    fi = f % NI
    pltpu.make_async_copy(
        adj_ref.at[fb, pl.ds(fi * BM, BM), :],
        ring_scr.at[f % RING],
        sem.at[f % RING],
    ).wait()


def _fused_kernel(adj_ref, x_ref, w1_ref, b1_ref, w2_ref, b2_ref,
                  h2_ref, gh_ref, s1_scr, s2_scr, cache_scr, ring_scr, sem):
    t = pl.program_id(0)
    b = t // (2 * NI)
    p = (t // NI) % 2
    i = t % NI
    c = b * NI + i

    @pl.when(t == 0)
    def _():
        for f0 in range(RING):
            _start_fetch(adj_ref, ring_scr, sem, f0)

    @pl.when(p == 0)
    def _():
        @pl.when(i == 0)
        def _():
            s1 = jnp.dot(
                x_ref[0], w1_ref[...], preferred_element_type=jnp.float32
            )
            s1_scr[...] = s1.astype(jnp.bfloat16)

        _wait_fetch(adj_ref, ring_scr, sem, c)
        a = ring_scr[c % RING].astype(jnp.bfloat16)
        cache_scr[pl.ds(i * BM, BM), :] = a

        @pl.when(c + RING < NF)
        def _():
            _start_fetch(adj_ref, ring_scr, sem, c + RING)

        tacc = jnp.dot(a, s1_scr[...], preferred_element_type=jnp.float32)
        h1 = jnp.maximum(tacc + b1_ref[...], 0.0)
        s2_scr[pl.ds(i * BM, BM), :] = jnp.dot(
            h1, w2_ref[...], preferred_element_type=jnp.float32
        ).astype(jnp.bfloat16)
        part = jnp.sum(h1, axis=0, keepdims=True)
        row = 2 * b

        @pl.when(i == 0)
        def _():
            gh_ref[pl.ds(row, 1), :] = part

        @pl.when(i != 0)
        def _():
            gh_ref[pl.ds(row, 1), :] += part

    @pl.when(p == 1)
    def _():
        a = cache_scr[pl.ds(i * BM, BM), :]
        tacc = jnp.dot(a, s2_scr[...], preferred_element_type=jnp.float32)
        h2 = jnp.maximum(tacc + b2_ref[...], 0.0)
        h2_ref[...] = h2[None]
        part = jnp.sum(h2, axis=0, keepdims=True)
        row = 2 * b + 1

        @pl.when(i == 0)
        def _():
            gh_ref[pl.ds(row, 1), :] = part

        @pl.when(i != 0)
        def _():
            gh_ref[pl.ds(row, 1), :] += part


@functools.partial(jax.jit, static_argnames=("interpret",))
def _encoder(x, adj, W1, b1, W2, b2, interpret=False):
    b1r = b1.reshape(1, H)
    b2r = b2.reshape(1, H)

    h2, gh = pl.pallas_call(
        _fused_kernel,
        grid=(2 * B * NI,),
        in_specs=[
            pl.BlockSpec(memory_space=pl.ANY),
            pl.BlockSpec((1, N, F), lambda t: (t // (2 * NI), 0, 0)),
            pl.BlockSpec((F, H), lambda t: (0, 0)),
            pl.BlockSpec((1, H), lambda t: (0, 0)),
            pl.BlockSpec((H, H), lambda t: (0, 0)),
            pl.BlockSpec((1, H), lambda t: (0, 0)),
        ],
        out_specs=[
            pl.BlockSpec(
                (1, BM, H),
                lambda t: (
                    t // (2 * NI),
                    jnp.where((t // NI) % 2 == 0, 0, t % NI),
                    0,
                ),
            ),
            pl.BlockSpec((4, H), lambda t: (0, 0)),
        ],
        out_shape=[
            jax.ShapeDtypeStruct((B, N, H), jnp.float32),
            jax.ShapeDtypeStruct((4, H), jnp.float32),
        ],
        scratch_shapes=[
            pltpu.VMEM((N, H), jnp.bfloat16),
            pltpu.VMEM((N, H), jnp.bfloat16),
            pltpu.VMEM((N, N), jnp.bfloat16),
            pltpu.VMEM((RING, BM, N), jnp.float32),
            pltpu.SemaphoreType.DMA((RING,)),
        ],
        compiler_params=pltpu.CompilerParams(
            dimension_semantics=("arbitrary",),
            vmem_limit_bytes=100 * 1024 * 1024,
        ),
        interpret=interpret,
    )(adj, x, W1, b1r, W2, b2r)

    return h2, gh.reshape(B, 2 * H)


def kernel(x, adj, W1, b1, W2, b2):
    return _encoder(x, adj, W1, b1, W2, b2)


# speedup vs baseline: 1.1706x; 1.0153x over previous
import functools

import jax
import jax.numpy as jnp
from jax.experimental import pallas as pl
from jax.experimental.pallas import tpu as pltpu

B, N, F, H = 2, 4096, 128, 128
BM = 512
NI = N // BM
RING = 3
NF = B * NI


KSPLIT = 4
SUB = BM // KSPLIT


def _sub_copy(adj_ref, ring_scr, sem, f, k):
    fb = f // NI
    fi = f % NI
    return pltpu.make_async_copy(
        adj_ref.at[fb, pl.ds(fi * BM + k * SUB, SUB), :],
        ring_scr.at[f % RING, pl.ds(k * SUB, SUB), :],
        sem.at[f % RING, k],
    )


def _start_fetch(adj_ref, ring_scr, sem, f):
    for k in range(KSPLIT):
        _sub_copy(adj_ref, ring_scr, sem, f, k).start()


def _wait_fetch(adj_ref, ring_scr, sem, f):
    for k in range(KSPLIT):
        _sub_copy(adj_ref, ring_scr, sem, f, k).wait()


def _fused_kernel(adj_ref, x_ref, w1_ref, b1_ref, w2_ref, b2_ref,
                  h2_ref, gh_ref, s1_scr, s2_scr, cache_scr, ring_scr, sem):
    t = pl.program_id(0)
    b = t // (2 * NI)
    p = (t // NI) % 2
    i = t % NI
    c = b * NI + i

    @pl.when(t == 0)
    def _():
        for f0 in range(RING):
            _start_fetch(adj_ref, ring_scr, sem, f0)

    @pl.when(p == 0)
    def _():
        @pl.when(i == 0)
        def _():
            s1 = jnp.dot(
                x_ref[0], w1_ref[...], preferred_element_type=jnp.float32
            )
            s1_scr[...] = s1.astype(jnp.bfloat16)

        _wait_fetch(adj_ref, ring_scr, sem, c)
        a = ring_scr[c % RING].astype(jnp.bfloat16)
        cache_scr[pl.ds(i * BM, BM), :] = a

        @pl.when(c + RING < NF)
        def _():
            _start_fetch(adj_ref, ring_scr, sem, c + RING)

        tacc = jnp.dot(a, s1_scr[...], preferred_element_type=jnp.float32)
        h1 = jnp.maximum(tacc + b1_ref[...], 0.0)
        s2_scr[pl.ds(i * BM, BM), :] = jnp.dot(
            h1, w2_ref[...], preferred_element_type=jnp.float32
        ).astype(jnp.bfloat16)
        part = jnp.sum(h1, axis=0, keepdims=True)
        row = 2 * b

        @pl.when(i == 0)
        def _():
            gh_ref[pl.ds(row, 1), :] = part

        @pl.when(i != 0)
        def _():
            gh_ref[pl.ds(row, 1), :] += part

    @pl.when(p == 1)
    def _():
        a = cache_scr[pl.ds(i * BM, BM), :]
        tacc = jnp.dot(a, s2_scr[...], preferred_element_type=jnp.float32)
        h2 = jnp.maximum(tacc + b2_ref[...], 0.0)
        h2_ref[...] = h2[None]
        part = jnp.sum(h2, axis=0, keepdims=True)
        row = 2 * b + 1

        @pl.when(i == 0)
        def _():
            gh_ref[pl.ds(row, 1), :] = part

        @pl.when(i != 0)
        def _():
            gh_ref[pl.ds(row, 1), :] += part


@functools.partial(jax.jit, static_argnames=("interpret",))
def _encoder(x, adj, W1, b1, W2, b2, interpret=False):
    b1r = b1.reshape(1, H)
    b2r = b2.reshape(1, H)

    h2, gh = pl.pallas_call(
        _fused_kernel,
        grid=(2 * B * NI,),
        in_specs=[
            pl.BlockSpec(memory_space=pl.ANY),
            pl.BlockSpec((1, N, F), lambda t: (t // (2 * NI), 0, 0)),
            pl.BlockSpec((F, H), lambda t: (0, 0)),
            pl.BlockSpec((1, H), lambda t: (0, 0)),
            pl.BlockSpec((H, H), lambda t: (0, 0)),
            pl.BlockSpec((1, H), lambda t: (0, 0)),
        ],
        out_specs=[
            pl.BlockSpec(
                (1, BM, H),
                lambda t: (
                    t // (2 * NI),
                    jnp.where((t // NI) % 2 == 0, 0, t % NI),
                    0,
                ),
            ),
            pl.BlockSpec((4, H), lambda t: (0, 0)),
        ],
        out_shape=[
            jax.ShapeDtypeStruct((B, N, H), jnp.float32),
            jax.ShapeDtypeStruct((4, H), jnp.float32),
        ],
        scratch_shapes=[
            pltpu.VMEM((N, H), jnp.bfloat16),
            pltpu.VMEM((N, H), jnp.bfloat16),
            pltpu.VMEM((N, N), jnp.bfloat16),
            pltpu.VMEM((RING, BM, N), jnp.float32),
            pltpu.SemaphoreType.DMA((RING, KSPLIT)),
        ],
        compiler_params=pltpu.CompilerParams(
            dimension_semantics=("arbitrary",),
            vmem_limit_bytes=100 * 1024 * 1024,
        ),
        interpret=interpret,
    )(adj, x, W1, b1r, W2, b2r)

    return h2, gh.reshape(B, 2 * H)


def kernel(x, adj, W1, b1, W2, b2):
    return _encoder(x, adj, W1, b1, W2, b2)


# speedup vs baseline: 1.2301x; 1.0508x over previous
import functools

import jax
import jax.numpy as jnp
from jax.experimental import pallas as pl
from jax.experimental.pallas import tpu as pltpu

B, N, F, H = 2, 4096, 128, 128
BM = 512
NI = N // BM


def _fused_kernel(adj_ref, x_ref, w1_ref, b1_ref, w2_ref, b2_ref,
                  h2_ref, gh_ref, s1_scr, s2_scr, cache_scr):
    t = pl.program_id(0)

    @pl.when(t >= NI)
    def _():
        pb = jnp.where(t < 2 * NI, 0, 1)
        pi = jnp.where(t < 2 * NI, t - NI, t - 2 * NI)
        a = cache_scr[pl.ds(pi * BM, BM), :]
        tacc = jnp.dot(a, s2_scr[pb], preferred_element_type=jnp.float32)
        h2 = jnp.maximum(tacc + b2_ref[...], 0.0)
        h2_ref[...] = h2[None]
        part = jnp.sum(h2, axis=0, keepdims=True)
        row = 2 * pb + 1

        @pl.when(pi == 0)
        def _():
            gh_ref[pl.ds(row, 1), :] = part

        @pl.when(pi != 0)
        def _():
            gh_ref[pl.ds(row, 1), :] += part

    @pl.when(t < 2 * NI)
    def _():
        fb = jnp.where(t < NI, 0, 1)
        fi = jnp.where(t < NI, t, t - NI)

        @pl.when(fi == 0)
        def _():
            s1 = jnp.dot(
                x_ref[0], w1_ref[...], preferred_element_type=jnp.float32
            )
            s1_scr[...] = s1.astype(jnp.bfloat16)

        a = adj_ref[0].astype(jnp.bfloat16)
        cache_scr[pl.ds(fi * BM, BM), :] = a
        tacc = jnp.dot(a, s1_scr[...], preferred_element_type=jnp.float32)
        h1 = jnp.maximum(tacc + b1_ref[...], 0.0)
        s2_scr[fb, pl.ds(fi * BM, BM), :] = jnp.dot(
            h1, w2_ref[...], preferred_element_type=jnp.float32
        ).astype(jnp.bfloat16)
        part = jnp.sum(h1, axis=0, keepdims=True)
        row = 2 * fb

        @pl.when(fi == 0)
        def _():
            gh_ref[pl.ds(row, 1), :] = part

        @pl.when(fi != 0)
        def _():
            gh_ref[pl.ds(row, 1), :] += part


@functools.partial(jax.jit, static_argnames=("interpret",))
def _encoder(x, adj, W1, b1, W2, b2, interpret=False):
    b1r = b1.reshape(1, H)
    b2r = b2.reshape(1, H)

    h2, gh = pl.pallas_call(
        _fused_kernel,
        grid=(3 * NI,),
        in_specs=[
            pl.BlockSpec(
                (1, BM, N),
                lambda t: (
                    jnp.where(t < NI, 0, 1),
                    jnp.where(t < 2 * NI, jnp.where(t < NI, t, t - NI), NI - 1),
                    0,
                ),
            ),
            pl.BlockSpec((1, N, F), lambda t: (jnp.where(t < NI, 0, 1), 0, 0)),
            pl.BlockSpec((F, H), lambda t: (0, 0)),
            pl.BlockSpec((1, H), lambda t: (0, 0)),
            pl.BlockSpec((H, H), lambda t: (0, 0)),
            pl.BlockSpec((1, H), lambda t: (0, 0)),
        ],
        out_specs=[
            pl.BlockSpec(
                (1, BM, H),
                lambda t: (
                    jnp.where(t < 2 * NI, 0, 1),
                    jnp.where(
                        t < NI, 0, jnp.where(t < 2 * NI, t - NI, t - 2 * NI)
                    ),
                    0,
                ),
            ),
            pl.BlockSpec((4, H), lambda t: (0, 0)),
        ],
        out_shape=[
            jax.ShapeDtypeStruct((B, N, H), jnp.float32),
            jax.ShapeDtypeStruct((4, H), jnp.float32),
        ],
        scratch_shapes=[
            pltpu.VMEM((N, H), jnp.bfloat16),
            pltpu.VMEM((B, N, H), jnp.bfloat16),
            pltpu.VMEM((N, N), jnp.bfloat16),
        ],
        compiler_params=pltpu.CompilerParams(
            dimension_semantics=("arbitrary",),
            vmem_limit_bytes=100 * 1024 * 1024,
        ),
        interpret=interpret,
    )(adj, x, W1, b1r, W2, b2r)

    return h2, gh.reshape(B, 2 * H)


def kernel(x, adj, W1, b1, W2, b2):
    return _encoder(x, adj, W1, b1, W2, b2)
